# speedup vs baseline: 1.2676x; 1.2676x over previous
import functools

import jax
import jax.numpy as jnp
from jax import lax
from jax.experimental import pallas as pl
from jax.experimental.pallas import tpu as pltpu


def _im2col_fill(im2_ref, src, c, hp, wp):
    npad = hp * wp
    im2_ref[...] = jnp.zeros(im2_ref.shape, im2_ref.dtype)
    for dy in range(3):
        for dx in range(3):
            o = (dy - 1) * wp + (dx - 1)
            r0 = (dy * 3 + dx) * c
            if o >= 0:
                im2_ref[r0:r0 + c, 0:npad - o] = src[:, o:npad]
            else:
                im2_ref[r0:r0 + c, -o:npad] = src[:, 0:npad + o]


def _pass_a(x_ref, mask_ref, w1_ref, ws_ref, y1_ref, skip_ref, stats_ref,
            im2_ref, *, cin, cout, hp, wp):
    x = x_ref[...]
    _im2col_fill(im2_ref, x, cin, hp, wp)
    y = jnp.dot(w1_ref[...], im2_ref[...], preferred_element_type=jnp.float32)
    y1_ref[...] = y.astype(y1_ref.dtype)
    skip_ref[...] = jnp.dot(
        ws_ref[...], x, preferred_element_type=jnp.float32
    ).astype(skip_ref.dtype)
    hm = y * mask_ref[...]
    stats_ref[:, 0:1] = jnp.sum(hm, axis=1, keepdims=True)
    stats_ref[:, 1:2] = jnp.sum(hm * hm, axis=1, keepdims=True)


def _pass_b(y1_ref, mask_ref, sc1_ref, sh1_ref, w2_ref, h2_ref, stats_ref,
            im2_ref, *, cout, hp, wp):
    h1 = jnp.maximum(sc1_ref[...] * y1_ref[...].astype(jnp.float32)
                     + sh1_ref[...], 0.0)
    h1 = h1 * mask_ref[...]
    _im2col_fill(im2_ref, h1.astype(im2_ref.dtype), cout, hp, wp)
    y = jnp.dot(w2_ref[...], im2_ref[...], preferred_element_type=jnp.float32)
    h2_ref[...] = y.astype(h2_ref.dtype)
    hm = y * mask_ref[...]
    stats_ref[:, 0:1] = jnp.sum(hm, axis=1, keepdims=True)
    stats_ref[:, 1:2] = jnp.sum(hm * hm, axis=1, keepdims=True)


def _pass_c(h2_ref, skip_ref, sc2_ref, sh2_ref, out_ref):
    out_ref[...] = jnp.maximum(
        sc2_ref[...] * h2_ref[...].astype(jnp.float32)
        + sh2_ref[...] + skip_ref[...].astype(jnp.float32), 0.0
    ).astype(out_ref.dtype)


def kernel(x, w1, b1, g1, be1, w2, b2, g2, be2, ws, bs, *, eps=1e-5):
    N, Cin, H, W = x.shape
    Cout = w1.shape[-1]
    Hp, Wp = H + 2, W + 2
    Npad = Hp * Wp
    cin_p = max(8, -(-Cin // 8) * 8)
    f32 = jnp.float32
    bf16 = jnp.bfloat16
    dtype = x.dtype

    xpad = jnp.pad(x, ((0, 0), (0, cin_p - Cin), (1, 1), (1, 1)))
    xpad = xpad.reshape(N, cin_p, Npad).astype(bf16)

    row = jnp.arange(Hp)[:, None]
    col = jnp.arange(Wp)[None, :]
    mask = (((row >= 1) & (row <= H) & (col >= 1) & (col <= W))
            .astype(f32).reshape(1, Npad))

    w1p = jnp.pad(w1, ((0, 0), (0, 0), (0, cin_p - Cin), (0, 0)))
    w1mat = jnp.transpose(w1p, (3, 0, 1, 2)).reshape(Cout, 9 * cin_p).astype(bf16)
    wsmat = jnp.pad(ws, ((0, cin_p - Cin), (0, 0))).T.astype(bf16)
    w2mat = jnp.transpose(w2, (3, 0, 1, 2)).reshape(Cout, 9 * Cout).astype(bf16)

    cparams = pltpu.CompilerParams(
        dimension_semantics=("parallel",),
        vmem_limit_bytes=48 * 1024 * 1024)

    def full(shape):
        return pl.BlockSpec(shape, lambda n: (0,) * len(shape))

    def per_n(r, c):
        return pl.BlockSpec((None, r, c), lambda n: (n, 0, 0))

    y1, skip, stats1 = pl.pallas_call(
        functools.partial(_pass_a, cin=cin_p, cout=Cout, hp=Hp, wp=Wp),
        grid=(N,),
        in_specs=[per_n(cin_p, Npad), full((1, Npad)),
                  full((Cout, 9 * cin_p)), full((Cout, cin_p))],
        out_specs=[per_n(Cout, Npad), per_n(Cout, Npad), per_n(Cout, 2)],
        out_shape=[jax.ShapeDtypeStruct((N, Cout, Npad), bf16),
                   jax.ShapeDtypeStruct((N, Cout, Npad), bf16),
                   jax.ShapeDtypeStruct((N, Cout, 2), f32)],
        scratch_shapes=[pltpu.VMEM((9 * cin_p, Npad), bf16)],
        compiler_params=cparams,
    )(xpad, mask, w1mat, wsmat)

    cnt = float(N * H * W)
    tot1 = jnp.sum(stats1, axis=0)
    mean1 = tot1[:, 0] / cnt
    var1 = jnp.maximum(tot1[:, 1] / cnt - mean1 * mean1, 0.0)
    sc1 = (g1.reshape(Cout) * lax.rsqrt(var1 + eps)).reshape(Cout, 1).astype(f32)
    sh1 = (be1.reshape(Cout) - sc1[:, 0] * mean1).reshape(Cout, 1).astype(f32)

    h2, stats2 = pl.pallas_call(
        functools.partial(_pass_b, cout=Cout, hp=Hp, wp=Wp),
        grid=(N,),
        in_specs=[per_n(Cout, Npad), full((1, Npad)), full((Cout, 1)),
                  full((Cout, 1)), full((Cout, 9 * Cout))],
        out_specs=[per_n(Cout, Npad), per_n(Cout, 2)],
        out_shape=[jax.ShapeDtypeStruct((N, Cout, Npad), bf16),
                   jax.ShapeDtypeStruct((N, Cout, 2), f32)],
        scratch_shapes=[pltpu.VMEM((9 * Cout, Npad), bf16)],
        compiler_params=cparams,
    )(y1, mask, sc1, sh1, w2mat)

    tot2 = jnp.sum(stats2, axis=0)
    mean2 = tot2[:, 0] / cnt
    var2 = jnp.maximum(tot2[:, 1] / cnt - mean2 * mean2, 0.0)
    sc2 = (g2.reshape(Cout) * lax.rsqrt(var2 + eps)).reshape(Cout, 1).astype(f32)
    sh2 = (be2.reshape(Cout) - sc2[:, 0] * mean2
           + bs.reshape(Cout)).reshape(Cout, 1).astype(f32)

    out_pad = pl.pallas_call(
        _pass_c,
        grid=(N,),
        in_specs=[per_n(Cout, Npad), per_n(Cout, Npad),
                  full((Cout, 1)), full((Cout, 1))],
        out_specs=per_n(Cout, Npad),
        out_shape=jax.ShapeDtypeStruct((N, Cout, Npad), dtype),
        compiler_params=cparams,
    )(h2, skip, sc2, sh2)

    return out_pad.reshape(N, Cout, Hp, Wp)[:, :, 1:H + 1, 1:W + 1]
